# baseline (speedup 1.0000x reference)
import functools

import jax
import jax.numpy as jnp
from jax import lax
from jax.experimental import pallas as pl
from jax.experimental.pallas import tpu as pltpu
from jax.experimental.pallas import tpu_sc as plsc

_K = 64
_N = 32768
_ROWS = 128
_NC = 2
_NS = 16
_NW = _NC * _NS
_RPW = _ROWS // _NW
_SEED_CHUNKS = 8
_GROUPS = (_N // 16 - _SEED_CHUNKS) // 8
_CKPTS = 15
_GPC = _GROUPS // _CKPTS
_PERCAP = 208
_CAP = 16 * _PERCAP
_TRIG_K = 72
_IMIN = -2147483648


def _sortable(v):
    k = lax.bitcast_convert_type(v, jnp.int32)
    return jnp.where(k >= 0, k, k ^ jnp.int32(0x7FFFFFFF))


def _unsortable(s):
    k = jnp.where(s >= 0, s, s ^ jnp.int32(0x7FFFFFFF))
    return lax.bitcast_convert_type(k, jnp.float32)


def _splat(x, dtype=jnp.int32):
    return lax.broadcast(jnp.asarray(x, dtype), (16,))


def _gather_sortable(buf, candi, sbuf, cursors, nv):
    iota16 = lax.iota(jnp.int32, 16)

    def g_body(vi, _):
        base = vi * 16
        posv = lax.broadcast(base, (16,)) + iota16
        valid = cursors > posv
        idxv = candi[pl.ds(base, 16)]
        vals = plsc.load_gather(buf, [idxv], mask=valid)
        sbuf[pl.ds(base, 16)] = jnp.where(valid, _sortable(vals),
                                          jnp.int32(_IMIN))
        return 0

    lax.fori_loop(0, nv, g_body, 0)


def _radix_select(sbuf, nv, nbits):
    zeros_i = jnp.zeros((16,), jnp.int32)

    def bit_body(i, p_u):
        cand_u = p_u | _splat(jnp.int32(1) << (31 - i))
        cand_s = cand_u ^ jnp.int32(_IMIN)

        def cnt_body(vi, acc):
            s = sbuf[pl.ds(vi * 16, 16)]
            return acc + plsc.all_reduce_population_count(s >= cand_s)

        cnt = lax.fori_loop(0, nv, cnt_body, zeros_i)
        return jnp.where(cnt >= _K, cand_u, p_u)

    p_u = lax.fori_loop(0, nbits, bit_body, zeros_i)
    return p_u ^ jnp.int32(_IMIN)


def _compact(candi, sbuf, keep_fn, nv):
    iota16 = lax.iota(jnp.int32, 16)
    c16 = _splat(16)

    def c_body(vi, newcur):
        base = vi * 16
        s = sbuf[pl.ds(base, 16)]
        idxv = candi[pl.ds(base, 16)]
        keep = keep_fn(s, idxv)
        plsc.store_scatter(candi, [newcur], idxv, mask=keep)
        return newcur + jnp.where(keep, c16, 0)

    return lax.fori_loop(0, nv, c_body, iota16)


def _mid_rebuild(buf, candi, sbuf, cursors, thresh):
    nv = jnp.max(cursors) >> 4
    _gather_sortable(buf, candi, sbuf, cursors, nv)
    ts = _radix_select(sbuf, nv, 16)
    newcur = _compact(candi, sbuf, lambda s, i: s >= ts, nv)
    return newcur, jnp.maximum(thresh, _unsortable(ts))


def _exact_rebuild(buf, candi, sbuf, cursors, thresh):
    zeros_i = jnp.zeros((16,), jnp.int32)
    nv = jnp.max(cursors) >> 4
    _gather_sortable(buf, candi, sbuf, cursors, nv)
    v64s = _radix_select(sbuf, nv, 32)

    def cnt_body(vi, carry):
        cg, ce = carry
        s = sbuf[pl.ds(vi * 16, 16)]
        return (cg + plsc.all_reduce_population_count(s > v64s),
                ce + plsc.all_reduce_population_count(s == v64s))

    c_gt, c_eq = lax.fori_loop(0, nv, cnt_body, (zeros_i, zeros_i))
    m_allow = _splat(_K) - c_gt

    def tie_radix(_):
        def bit_body(i, carry):
            p, m_rem = carry
            w = _splat(jnp.int32(1) << (14 - i))

            def c_body(vi, acc):
                base = vi * 16
                s = sbuf[pl.ds(base, 16)]
                idxv = candi[pl.ds(base, 16)]
                hit = (s == v64s) & (idxv >= p) & (idxv < p + w)
                return acc + plsc.all_reduce_population_count(hit)

            c0 = lax.fori_loop(0, nv, c_body, zeros_i)
            low = m_rem <= c0
            return jnp.where(low, p, p + w), jnp.where(low, m_rem,
                                                       m_rem - c0)

        p, _ = lax.fori_loop(0, 15, bit_body, (zeros_i, m_allow))
        return p

    tie_j = lax.cond(jnp.max(c_eq) == jnp.max(m_allow),
                     lambda _: _splat(_N), tie_radix, 0)

    newcur = _compact(
        candi, sbuf,
        lambda s, i: (s > v64s) | ((s == v64s) & (i <= tie_j)), nv)
    return newcur, jnp.maximum(thresh, _unsortable(v64s))


def _scan_row(buf, candi, sbuf):
    iota16 = lax.iota(jnp.int32, 16)
    c16 = _splat(16)

    cursors = iota16
    for c in range(_SEED_CHUNKS):
        idxv = iota16 + _splat(c * 16)
        plsc.store_scatter(candi, [cursors], idxv)
        cursors = cursors + c16
    thresh0 = jnp.full((16,), -jnp.inf, jnp.float32)
    cursors, thresh0 = _mid_rebuild(buf, candi, sbuf, cursors, thresh0)

    def ckpt_body(kc, carry):
        cursors, thresh, idxb = carry

        def group_body(g, c):
            cur, ib = c
            base = _SEED_CHUNKS * 16 + g * 128
            for u in range(8):
                v = buf[pl.ds(base + u * 16, 16)]
                m = v > thresh
                plsc.store_scatter(candi, [cur], ib, mask=m)
                cur = cur + jnp.where(m, c16, 0)
                ib = ib + c16
            return cur, ib

        cursors, idxb = lax.fori_loop(kc * _GPC, (kc + 1) * _GPC,
                                      group_body, (cursors, idxb))

        def do_rebuild(c):
            cur, th = c
            cur, th = _mid_rebuild(buf, candi, sbuf, cur, th)
            return lax.cond(
                jnp.max(cur) >> 4 >= _TRIG_K,
                lambda cc: _exact_rebuild(buf, candi, sbuf, cc[0], cc[1]),
                lambda cc: cc,
                (cur, th))

        cursors, thresh = lax.cond(jnp.max(cursors) >> 4 >= _TRIG_K,
                                   do_rebuild, lambda c: c,
                                   (cursors, thresh))
        return cursors, thresh, idxb

    idxb0 = iota16 + _splat(_SEED_CHUNKS * 16)
    cursors, thresh, _ = lax.fori_loop(0, _CKPTS, ckpt_body,
                                       (cursors, thresh0, idxb0))
    cursors, _ = _exact_rebuild(buf, candi, sbuf, cursors, thresh)
    return cursors


_mesh = plsc.VectorSubcoreMesh(core_axis_name="c", subcore_axis_name="s")

_KERNEL_KWARGS = dict(
    mesh=_mesh,
    compiler_params=pltpu.CompilerParams(needs_layout_passes=False),
    out_type=jax.ShapeDtypeStruct((_ROWS, _N), jnp.float32),
    scratch_types=[
        pltpu.VMEM((_N,), jnp.float32),
        pltpu.VMEM((_N,), jnp.float32),
        pltpu.VMEM((_N,), jnp.float32),
        pltpu.VMEM((_CAP,), jnp.int32),
        pltpu.VMEM((_CAP,), jnp.int32),
        pltpu.VMEM((_K * 16,), jnp.int32),
        pltpu.VMEM((16,), jnp.int32),
        pltpu.SemaphoreType.DMA,
        pltpu.SemaphoreType.DMA,
        pltpu.SemaphoreType.DMA,
    ],
)


def _sc_topk_body(x_hbm, out_hbm, rowbuf_a, rowbuf_b, outbuf, candi, sbuf,
                  previdx, prevcur, sem_a, sem_b, sem_o):
    iota16 = lax.iota(jnp.int32, 16)
    wid = lax.axis_index("s") * _NC + lax.axis_index("c")
    r0 = wid * _RPW
    zf16 = jnp.zeros((16,), jnp.float32)

    def z_body(i, _):
        for u in range(8):
            outbuf[pl.ds(i * 128 + u * 16, 16)] = zf16
        return 0

    lax.fori_loop(0, _N // 128, z_body, 0)

    sems = (sem_a, sem_b)
    bufs = (rowbuf_a, rowbuf_b)
    pltpu.make_async_copy(x_hbm.at[r0], rowbuf_a, sem_a).start()
    for j in range(_RPW):
        rj = r0 + j
        buf = bufs[j % 2]
        pltpu.make_async_copy(x_hbm.at[rj], buf, sems[j % 2]).wait()
        if j + 1 < _RPW:
            pltpu.make_async_copy(x_hbm.at[rj + 1], bufs[(j + 1) % 2],
                                  sems[(j + 1) % 2]).start()

        cursors = _scan_row(buf, candi, sbuf)
        nvf = jnp.max(cursors) >> 4

        if j > 0:
            pltpu.make_async_copy(outbuf, out_hbm.at[rj - 1], sem_o).wait()
            pcur = prevcur[pl.ds(0, 16)]

            def unsc_body(vi, _):
                base = vi * 16
                posv = lax.broadcast(base, (16,)) + iota16
                valid = pcur > posv
                idxv = previdx[pl.ds(base, 16)]
                plsc.store_scatter(outbuf, [idxv], zf16, mask=valid)
                return 0

            lax.fori_loop(0, jnp.max(pcur) >> 4, unsc_body, 0)

        def sc_body(vi, _):
            base = vi * 16
            posv = lax.broadcast(base, (16,)) + iota16
            valid = cursors > posv
            idxv = candi[pl.ds(base, 16)]
            vals = plsc.load_gather(buf, [idxv], mask=valid)
            plsc.store_scatter(outbuf, [idxv], jnp.maximum(vals, 0.0),
                               mask=valid)
            previdx[pl.ds(base, 16)] = idxv
            return 0

        lax.fori_loop(0, nvf, sc_body, 0)
        prevcur[pl.ds(0, 16)] = cursors
        pltpu.make_async_copy(outbuf, out_hbm.at[rj], sem_o).start()

    pltpu.make_async_copy(outbuf, out_hbm.at[r0 + _RPW - 1], sem_o).wait()


_sc_topk = pl.kernel(_sc_topk_body, **_KERNEL_KWARGS)


def kernel(x):
    return _sc_topk(x)


# speedup vs baseline: 1.5107x; 1.5107x over previous
import functools

import jax
import jax.numpy as jnp
from jax import lax
from jax.experimental import pallas as pl
from jax.experimental.pallas import tpu as pltpu
from jax.experimental.pallas import tpu_sc as plsc

_K = 64
_N = 32768
_ROWS = 128
_NC = 2
_NS = 16
_NW = _NC * _NS
_RPW = _ROWS // _NW
_SEED_CHUNKS = 8
_GROUPS = (_N // 16 - _SEED_CHUNKS) // 8
_CKPTS = 15
_GPC = _GROUPS // _CKPTS
_PERCAP = 208
_CAP = 16 * _PERCAP
_TRIG_K = 72
_IMIN = -2147483648


def _sortable(v):
    k = lax.bitcast_convert_type(v, jnp.int32)
    return jnp.where(k >= 0, k, k ^ jnp.int32(0x7FFFFFFF))


def _unsortable(s):
    k = jnp.where(s >= 0, s, s ^ jnp.int32(0x7FFFFFFF))
    return lax.bitcast_convert_type(k, jnp.float32)


def _splat(x, dtype=jnp.int32):
    return lax.broadcast(jnp.asarray(x, dtype), (16,))


def _gather_sortable(buf, candi, sbuf, cursors, nv4):
    iota16 = lax.iota(jnp.int32, 16)

    def g_body(vi4, _):
        for u in range(4):
            base = (vi4 * 4 + u) * 16
            posv = lax.broadcast(base, (16,)) + iota16
            valid = cursors > posv
            idxv = candi[pl.ds(base, 16)]
            vals = plsc.load_gather(buf, [idxv], mask=valid)
            sbuf[pl.ds(base, 16)] = jnp.where(valid, _sortable(vals),
                                              jnp.int32(_IMIN))
        return 0

    lax.fori_loop(0, nv4, g_body, 0)


def _radix_select(sbuf, nv4, nbits):
    zeros_i = jnp.zeros((16,), jnp.int32)

    def bit_body(i, p_u):
        cand_u = p_u | _splat(jnp.int32(1) << (31 - i))
        cand_s = cand_u ^ jnp.int32(_IMIN)

        def cnt_body(vi4, acc):
            for u in range(4):
                s = sbuf[pl.ds((vi4 * 4 + u) * 16, 16)]
                acc = acc + plsc.all_reduce_population_count(s >= cand_s)
            return acc

        cnt = lax.fori_loop(0, nv4, cnt_body, zeros_i)
        return jnp.where(cnt >= _K, cand_u, p_u)

    p_u = lax.fori_loop(0, nbits, bit_body, zeros_i)
    return p_u ^ jnp.int32(_IMIN)


def _compact(candi, sbuf, keep_fn, nv4):
    iota16 = lax.iota(jnp.int32, 16)
    c16 = _splat(16)

    def c_body(vi4, newcur):
        for u in range(4):
            base = (vi4 * 4 + u) * 16
            s = sbuf[pl.ds(base, 16)]
            idxv = candi[pl.ds(base, 16)]
            keep = keep_fn(s, idxv)
            plsc.store_scatter(candi, [newcur], idxv, mask=keep)
            newcur = newcur + jnp.where(keep, c16, 0)
        return newcur

    return lax.fori_loop(0, nv4, c_body, iota16)


def _mid_rebuild(buf, candi, sbuf, cursors, thresh):
    iota16 = lax.iota(jnp.int32, 16)
    c16 = _splat(16)
    neg_inf = jnp.full((16,), -jnp.inf, jnp.float32)
    nv = jnp.max(cursors) >> 4

    def p1_body(vi, carry):
        m1, m2, m3, m4 = carry
        base = vi * 16
        posv = lax.broadcast(base, (16,)) + iota16
        valid = cursors > posv
        idxv = candi[pl.ds(base, 16)]
        vals = plsc.load_gather(buf, [idxv], mask=valid)
        v = jnp.where(valid, vals, neg_inf)
        sbuf[pl.ds(base, 16)] = lax.bitcast_convert_type(v, jnp.int32)
        h1 = jnp.maximum(m1, v); v = jnp.minimum(m1, v)
        h2 = jnp.maximum(m2, v); v = jnp.minimum(m2, v)
        h3 = jnp.maximum(m3, v); v = jnp.minimum(m3, v)
        h4 = jnp.maximum(m4, v)
        return h1, h2, h3, h4

    _, _, _, m4 = lax.fori_loop(0, nv, p1_body,
                                (neg_inf, neg_inf, neg_inf, neg_inf))
    ts = lax.broadcast(jnp.min(m4), (16,))

    def p2_body(vi, newcur):
        base = vi * 16
        posv = lax.broadcast(base, (16,)) + iota16
        valid = cursors > posv
        v = lax.bitcast_convert_type(sbuf[pl.ds(base, 16)], jnp.float32)
        idxv = candi[pl.ds(base, 16)]
        keep = (v >= ts) & valid
        plsc.store_scatter(candi, [newcur], idxv, mask=keep)
        return newcur + jnp.where(keep, c16, 0)

    newcur = lax.fori_loop(0, nv, p2_body, iota16)
    return newcur, jnp.maximum(thresh, ts)


def _exact_rebuild(buf, candi, sbuf, cursors, thresh):
    zeros_i = jnp.zeros((16,), jnp.int32)
    nv4 = ((jnp.max(cursors) >> 4) + 3) >> 2
    _gather_sortable(buf, candi, sbuf, cursors, nv4)
    v64s = _radix_select(sbuf, nv4, 32)

    def cnt_body(vi4, carry):
        cg, ce = carry
        for u in range(4):
            s = sbuf[pl.ds((vi4 * 4 + u) * 16, 16)]
            cg = cg + plsc.all_reduce_population_count(s > v64s)
            ce = ce + plsc.all_reduce_population_count(s == v64s)
        return cg, ce

    c_gt, c_eq = lax.fori_loop(0, nv4, cnt_body, (zeros_i, zeros_i))
    m_allow = _splat(_K) - c_gt

    def tie_radix(_):
        def bit_body(i, carry):
            p, m_rem = carry
            w = _splat(jnp.int32(1) << (14 - i))

            def c_body(vi4, acc):
                for u in range(4):
                    base = (vi4 * 4 + u) * 16
                    s = sbuf[pl.ds(base, 16)]
                    idxv = candi[pl.ds(base, 16)]
                    hit = (s == v64s) & (idxv >= p) & (idxv < p + w)
                    acc = acc + plsc.all_reduce_population_count(hit)
                return acc

            c0 = lax.fori_loop(0, nv4, c_body, zeros_i)
            low = m_rem <= c0
            return jnp.where(low, p, p + w), jnp.where(low, m_rem,
                                                       m_rem - c0)

        p, _ = lax.fori_loop(0, 15, bit_body, (zeros_i, m_allow))
        return p

    tie_j = lax.cond(jnp.max(c_eq) == jnp.max(m_allow),
                     lambda _: _splat(_N), tie_radix, 0)

    newcur = _compact(
        candi, sbuf,
        lambda s, i: (s > v64s) | ((s == v64s) & (i <= tie_j)), nv4)
    return newcur, jnp.maximum(thresh, _unsortable(v64s))


def _scan_row(buf, candi, sbuf):
    iota16 = lax.iota(jnp.int32, 16)
    c16 = _splat(16)

    cursors = iota16
    for c in range(_SEED_CHUNKS):
        idxv = iota16 + _splat(c * 16)
        plsc.store_scatter(candi, [cursors], idxv)
        cursors = cursors + c16
    thresh0 = jnp.full((16,), -jnp.inf, jnp.float32)
    cursors, thresh0 = _mid_rebuild(buf, candi, sbuf, cursors, thresh0)

    def ckpt_body(kc, carry):
        cursors, thresh, idxb = carry

        def group_body(g, c):
            cur, ib = c
            base = _SEED_CHUNKS * 16 + g * 128
            for u in range(8):
                v = buf[pl.ds(base + u * 16, 16)]
                m = v > thresh
                plsc.store_scatter(candi, [cur], ib, mask=m)
                cur = cur + jnp.where(m, c16, 0)
                ib = ib + c16
            return cur, ib

        cursors, idxb = lax.fori_loop(kc * _GPC, (kc + 1) * _GPC,
                                      group_body, (cursors, idxb))

        def do_rebuild(c):
            cur, th = c
            cur, th = _mid_rebuild(buf, candi, sbuf, cur, th)
            return lax.cond(
                jnp.max(cur) >> 4 >= _TRIG_K,
                lambda cc: _exact_rebuild(buf, candi, sbuf, cc[0], cc[1]),
                lambda cc: cc,
                (cur, th))

        cursors, thresh = lax.cond(jnp.max(cursors) >> 4 >= _TRIG_K,
                                   do_rebuild, lambda c: c,
                                   (cursors, thresh))
        return cursors, thresh, idxb

    idxb0 = iota16 + _splat(_SEED_CHUNKS * 16)
    cursors, thresh, _ = lax.fori_loop(0, _CKPTS, ckpt_body,
                                       (cursors, thresh0, idxb0))
    cursors, _ = _exact_rebuild(buf, candi, sbuf, cursors, thresh)
    return cursors


_mesh = plsc.VectorSubcoreMesh(core_axis_name="c", subcore_axis_name="s")

_KERNEL_KWARGS = dict(
    mesh=_mesh,
    compiler_params=pltpu.CompilerParams(needs_layout_passes=False),
    out_type=jax.ShapeDtypeStruct((_ROWS, _N), jnp.float32),
    scratch_types=[
        pltpu.VMEM((_N,), jnp.float32),
        pltpu.VMEM((_N,), jnp.float32),
        pltpu.VMEM((_N,), jnp.float32),
        pltpu.VMEM((_CAP,), jnp.int32),
        pltpu.VMEM((_CAP,), jnp.int32),
        pltpu.VMEM((_K * 16,), jnp.int32),
        pltpu.VMEM((16,), jnp.int32),
        pltpu.SemaphoreType.DMA,
        pltpu.SemaphoreType.DMA,
        pltpu.SemaphoreType.DMA,
    ],
)


def _sc_topk_body(x_hbm, out_hbm, rowbuf_a, rowbuf_b, outbuf, candi, sbuf,
                  previdx, prevcur, sem_a, sem_b, sem_o):
    iota16 = lax.iota(jnp.int32, 16)
    wid = lax.axis_index("s") * _NC + lax.axis_index("c")
    r0 = wid * _RPW
    zf16 = jnp.zeros((16,), jnp.float32)

    def z_body(i, _):
        for u in range(8):
            outbuf[pl.ds(i * 128 + u * 16, 16)] = zf16
        return 0

    lax.fori_loop(0, _N // 128, z_body, 0)

    sems = (sem_a, sem_b)
    bufs = (rowbuf_a, rowbuf_b)
    pltpu.make_async_copy(x_hbm.at[r0], rowbuf_a, sem_a).start()
    for j in range(_RPW):
        rj = r0 + j
        buf = bufs[j % 2]
        pltpu.make_async_copy(x_hbm.at[rj], buf, sems[j % 2]).wait()
        if j + 1 < _RPW:
            pltpu.make_async_copy(x_hbm.at[rj + 1], bufs[(j + 1) % 2],
                                  sems[(j + 1) % 2]).start()

        cursors = _scan_row(buf, candi, sbuf)
        nvf = jnp.max(cursors) >> 4

        if j > 0:
            pltpu.make_async_copy(outbuf, out_hbm.at[rj - 1], sem_o).wait()
            pcur = prevcur[pl.ds(0, 16)]

            def unsc_body(vi, _):
                base = vi * 16
                posv = lax.broadcast(base, (16,)) + iota16
                valid = pcur > posv
                idxv = previdx[pl.ds(base, 16)]
                plsc.store_scatter(outbuf, [idxv], zf16, mask=valid)
                return 0

            lax.fori_loop(0, jnp.max(pcur) >> 4, unsc_body, 0)

        def sc_body(vi, _):
            base = vi * 16
            posv = lax.broadcast(base, (16,)) + iota16
            valid = cursors > posv
            idxv = candi[pl.ds(base, 16)]
            vals = plsc.load_gather(buf, [idxv], mask=valid)
            plsc.store_scatter(outbuf, [idxv], jnp.maximum(vals, 0.0),
                               mask=valid)
            previdx[pl.ds(base, 16)] = idxv
            return 0

        lax.fori_loop(0, nvf, sc_body, 0)
        prevcur[pl.ds(0, 16)] = cursors
        pltpu.make_async_copy(outbuf, out_hbm.at[rj], sem_o).start()

    pltpu.make_async_copy(outbuf, out_hbm.at[r0 + _RPW - 1], sem_o).wait()


_sc_topk = pl.kernel(_sc_topk_body, **_KERNEL_KWARGS)


def kernel(x):
    return _sc_topk(x)


# speedup vs baseline: 2.8592x; 1.8926x over previous
import functools

import jax
import jax.numpy as jnp
from jax import lax
from jax.experimental import pallas as pl
from jax.experimental.pallas import tpu as pltpu
from jax.experimental.pallas import tpu_sc as plsc

_K = 64
_N = 32768
_ROWS = 128
_NC = 2
_NS = 16
_NW = _NC * _NS
_RPW = _ROWS // _NW
_SEED_CHUNKS = 8
_GROUPS = (_N // 16 - _SEED_CHUNKS) // 8
_CKPTS = 15
_GPC = _GROUPS // _CKPTS
_PERCAP = 208
_CAP = 16 * _PERCAP
_TRIG_K = 72
_IMIN = -2147483648


def _sortable(v):
    k = lax.bitcast_convert_type(v, jnp.int32)
    return jnp.where(k >= 0, k, k ^ jnp.int32(0x7FFFFFFF))


def _unsortable(s):
    k = jnp.where(s >= 0, s, s ^ jnp.int32(0x7FFFFFFF))
    return lax.bitcast_convert_type(k, jnp.float32)


def _splat(x, dtype=jnp.int32):
    return lax.broadcast(jnp.asarray(x, dtype), (16,))


def _gather_sortable(buf, candi, sbuf, cursors, nv4):
    iota16 = lax.iota(jnp.int32, 16)

    def g_body(vi4, _):
        outs = []
        for u in range(4):
            base = (vi4 * 4 + u) * 16
            posv = lax.broadcast(base, (16,)) + iota16
            valid = cursors > posv
            idxv = candi[pl.ds(base, 16)]
            vals = plsc.load_gather(buf, [idxv], mask=valid)
            outs.append(jnp.where(valid, _sortable(vals),
                                  jnp.int32(_IMIN)))
        for u in range(4):
            sbuf[pl.ds((vi4 * 4 + u) * 16, 16)] = outs[u]
        return 0

    lax.fori_loop(0, nv4, g_body, 0)


def _radix_select(sbuf, nv4, nbits):
    zeros_i = jnp.zeros((16,), jnp.int32)

    def bit_body(i, p_u):
        cand_u = p_u | _splat(jnp.int32(1) << (31 - i))
        cand_s = cand_u ^ jnp.int32(_IMIN)

        def cnt_body(vi4, acc):
            for u in range(4):
                s = sbuf[pl.ds((vi4 * 4 + u) * 16, 16)]
                acc = acc + plsc.all_reduce_population_count(s >= cand_s)
            return acc

        cnt = lax.fori_loop(0, nv4, cnt_body, zeros_i)
        return jnp.where(cnt >= _K, cand_u, p_u)

    p_u = lax.fori_loop(0, nbits, bit_body, zeros_i)
    return p_u ^ jnp.int32(_IMIN)


def _compact(candi, sbuf, keep_fn, nv4):
    iota16 = lax.iota(jnp.int32, 16)
    c16 = _splat(16)

    def c_body(vi4, newcur):
        ks, ivs = [], []
        for u in range(4):
            base = (vi4 * 4 + u) * 16
            s = sbuf[pl.ds(base, 16)]
            idxv = candi[pl.ds(base, 16)]
            ivs.append(idxv)
            ks.append(keep_fn(s, idxv))
        sels = [jnp.where(k, c16, 0) for k in ks]
        curs = [newcur, newcur + sels[0], newcur + sels[0] + sels[1],
                newcur + sels[0] + sels[1] + sels[2]]
        for u in range(4):
            plsc.store_scatter(candi, [curs[u]], ivs[u], mask=ks[u])
        return curs[3] + sels[3]

    return lax.fori_loop(0, nv4, c_body, iota16)


def _mid_rebuild(buf, candi, sbuf, cursors, thresh):
    iota16 = lax.iota(jnp.int32, 16)
    c16 = _splat(16)
    neg_inf = jnp.full((16,), -jnp.inf, jnp.float32)
    nv2 = ((jnp.max(cursors) >> 4) + 1) >> 1

    def _ins(acc, v):
        m1, m2, m3, m4 = acc
        h1 = jnp.maximum(m1, v); v = jnp.minimum(m1, v)
        h2 = jnp.maximum(m2, v); v = jnp.minimum(m2, v)
        h3 = jnp.maximum(m3, v); v = jnp.minimum(m3, v)
        return h1, h2, h3, jnp.maximum(m4, v)

    def p1_body(vi2, carry):
        acc_a, acc_b = carry
        vs = []
        for u in range(2):
            base = (vi2 * 2 + u) * 16
            posv = lax.broadcast(base, (16,)) + iota16
            valid = cursors > posv
            idxv = candi[pl.ds(base, 16)]
            vals = plsc.load_gather(buf, [idxv], mask=valid)
            vs.append(jnp.where(valid, vals, neg_inf))
        for u in range(2):
            sbuf[pl.ds((vi2 * 2 + u) * 16, 16)] = lax.bitcast_convert_type(
                vs[u], jnp.int32)
        return _ins(acc_a, vs[0]), _ins(acc_b, vs[1])

    init = (neg_inf, neg_inf, neg_inf, neg_inf)
    acc_a, acc_b = lax.fori_loop(0, nv2, p1_body, (init, init))
    for v in acc_b:
        acc_a = _ins(acc_a, v)
    ts = lax.broadcast(jnp.min(acc_a[3]), (16,))

    def p2_body(vi2, newcur):
        ks, ivs = [], []
        for u in range(2):
            base = (vi2 * 2 + u) * 16
            posv = lax.broadcast(base, (16,)) + iota16
            valid = cursors > posv
            v = lax.bitcast_convert_type(sbuf[pl.ds(base, 16)], jnp.float32)
            ivs.append(candi[pl.ds(base, 16)])
            ks.append((v >= ts) & valid)
        cur1 = newcur + jnp.where(ks[0], c16, 0)
        plsc.store_scatter(candi, [newcur], ivs[0], mask=ks[0])
        plsc.store_scatter(candi, [cur1], ivs[1], mask=ks[1])
        return cur1 + jnp.where(ks[1], c16, 0)

    newcur = lax.fori_loop(0, nv2, p2_body, iota16)
    return newcur, jnp.maximum(thresh, ts)


def _exact_rebuild(buf, candi, sbuf, cursors, thresh):
    zeros_i = jnp.zeros((16,), jnp.int32)
    nv4 = ((jnp.max(cursors) >> 4) + 3) >> 2
    _gather_sortable(buf, candi, sbuf, cursors, nv4)
    v64s = _radix_select(sbuf, nv4, 32)

    def cnt_body(vi4, carry):
        cg, ce = carry
        for u in range(4):
            s = sbuf[pl.ds((vi4 * 4 + u) * 16, 16)]
            cg = cg + plsc.all_reduce_population_count(s > v64s)
            ce = ce + plsc.all_reduce_population_count(s == v64s)
        return cg, ce

    c_gt, c_eq = lax.fori_loop(0, nv4, cnt_body, (zeros_i, zeros_i))
    m_allow = _splat(_K) - c_gt

    def tie_radix(_):
        def bit_body(i, carry):
            p, m_rem = carry
            w = _splat(jnp.int32(1) << (14 - i))

            def c_body(vi4, acc):
                for u in range(4):
                    base = (vi4 * 4 + u) * 16
                    s = sbuf[pl.ds(base, 16)]
                    idxv = candi[pl.ds(base, 16)]
                    hit = (s == v64s) & (idxv >= p) & (idxv < p + w)
                    acc = acc + plsc.all_reduce_population_count(hit)
                return acc

            c0 = lax.fori_loop(0, nv4, c_body, zeros_i)
            low = m_rem <= c0
            return jnp.where(low, p, p + w), jnp.where(low, m_rem,
                                                       m_rem - c0)

        p, _ = lax.fori_loop(0, 15, bit_body, (zeros_i, m_allow))
        return p

    tie_j = lax.cond(jnp.max(c_eq) == jnp.max(m_allow),
                     lambda _: _splat(_N), tie_radix, 0)

    newcur = _compact(
        candi, sbuf,
        lambda s, i: (s > v64s) | ((s == v64s) & (i <= tie_j)), nv4)
    return newcur, jnp.maximum(thresh, _unsortable(v64s))


def _scan_row(buf, candi, sbuf):
    iota16 = lax.iota(jnp.int32, 16)
    c16 = _splat(16)

    cursors = iota16
    for c in range(_SEED_CHUNKS):
        idxv = iota16 + _splat(c * 16)
        plsc.store_scatter(candi, [cursors], idxv)
        cursors = cursors + c16
    thresh0 = jnp.full((16,), -jnp.inf, jnp.float32)
    cursors, thresh0 = _mid_rebuild(buf, candi, sbuf, cursors, thresh0)

    def ckpt_body(kc, carry):
        cursors, thresh, idxb = carry

        def group_body(g, c):
            cur, ib = c
            base = _SEED_CHUNKS * 16 + g * 128
            ms = [buf[pl.ds(base + u * 16, 16)] > thresh for u in range(8)]
            sels = [jnp.where(m, c16, 0) for m in ms]
            s01 = sels[0] + sels[1]
            s23 = sels[2] + sels[3]
            s45 = sels[4] + sels[5]
            s67 = sels[6] + sels[7]
            s03 = s01 + s23
            curs = [cur, cur + sels[0], cur + s01, cur + s01 + sels[2],
                    cur + s03, cur + s03 + sels[4], cur + s03 + s45,
                    cur + s03 + s45 + sels[6]]
            ibs = [ib] + [ib + _splat(16 * u) for u in range(1, 8)]
            for u in range(8):
                plsc.store_scatter(candi, [curs[u]], ibs[u], mask=ms[u])
            return curs[7] + sels[7], ib + _splat(128)

        cursors, idxb = lax.fori_loop(kc * _GPC, (kc + 1) * _GPC,
                                      group_body, (cursors, idxb))

        def do_rebuild(c):
            cur, th = c
            cur, th = _mid_rebuild(buf, candi, sbuf, cur, th)
            return lax.cond(
                jnp.max(cur) >> 4 >= _TRIG_K,
                lambda cc: _exact_rebuild(buf, candi, sbuf, cc[0], cc[1]),
                lambda cc: cc,
                (cur, th))

        cursors, thresh = lax.cond(jnp.max(cursors) >> 4 >= _TRIG_K,
                                   do_rebuild, lambda c: c,
                                   (cursors, thresh))
        return cursors, thresh, idxb

    idxb0 = iota16 + _splat(_SEED_CHUNKS * 16)
    cursors, thresh, _ = lax.fori_loop(0, _CKPTS, ckpt_body,
                                       (cursors, thresh0, idxb0))
    cursors, _ = _exact_rebuild(buf, candi, sbuf, cursors, thresh)
    return cursors


_mesh = plsc.VectorSubcoreMesh(core_axis_name="c", subcore_axis_name="s")

_KERNEL_KWARGS = dict(
    mesh=_mesh,
    compiler_params=pltpu.CompilerParams(needs_layout_passes=False),
    out_type=jax.ShapeDtypeStruct((_ROWS, _N), jnp.float32),
    scratch_types=[
        pltpu.VMEM((_N,), jnp.float32),
        pltpu.VMEM((_N,), jnp.float32),
        pltpu.VMEM((_N,), jnp.float32),
        pltpu.VMEM((_CAP,), jnp.int32),
        pltpu.VMEM((_CAP,), jnp.int32),
        pltpu.VMEM((_K * 16,), jnp.int32),
        pltpu.VMEM((16,), jnp.int32),
        pltpu.SemaphoreType.DMA,
        pltpu.SemaphoreType.DMA,
        pltpu.SemaphoreType.DMA,
    ],
)


def _sc_topk_body(x_hbm, out_hbm, rowbuf_a, rowbuf_b, outbuf, candi, sbuf,
                  previdx, prevcur, sem_a, sem_b, sem_o):
    iota16 = lax.iota(jnp.int32, 16)
    wid = lax.axis_index("s") * _NC + lax.axis_index("c")
    r0 = wid * _RPW
    zf16 = jnp.zeros((16,), jnp.float32)

    def z_body(i, _):
        for u in range(8):
            outbuf[pl.ds(i * 128 + u * 16, 16)] = zf16
        return 0

    lax.fori_loop(0, _N // 128, z_body, 0)

    sems = (sem_a, sem_b)
    bufs = (rowbuf_a, rowbuf_b)
    pltpu.make_async_copy(x_hbm.at[r0], rowbuf_a, sem_a).start()
    for j in range(_RPW):
        rj = r0 + j
        buf = bufs[j % 2]
        pltpu.make_async_copy(x_hbm.at[rj], buf, sems[j % 2]).wait()
        if j + 1 < _RPW:
            pltpu.make_async_copy(x_hbm.at[rj + 1], bufs[(j + 1) % 2],
                                  sems[(j + 1) % 2]).start()

        cursors = _scan_row(buf, candi, sbuf)
        nvf = jnp.max(cursors) >> 4

        if j > 0:
            pltpu.make_async_copy(outbuf, out_hbm.at[rj - 1], sem_o).wait()
            pcur = prevcur[pl.ds(0, 16)]

            def unsc_body(vi, _):
                base = vi * 16
                posv = lax.broadcast(base, (16,)) + iota16
                valid = pcur > posv
                idxv = previdx[pl.ds(base, 16)]
                plsc.store_scatter(outbuf, [idxv], zf16, mask=valid)
                return 0

            lax.fori_loop(0, jnp.max(pcur) >> 4, unsc_body, 0)

        def sc_body(vi, _):
            base = vi * 16
            posv = lax.broadcast(base, (16,)) + iota16
            valid = cursors > posv
            idxv = candi[pl.ds(base, 16)]
            vals = plsc.load_gather(buf, [idxv], mask=valid)
            plsc.store_scatter(outbuf, [idxv], jnp.maximum(vals, 0.0),
                               mask=valid)
            previdx[pl.ds(base, 16)] = idxv
            return 0

        lax.fori_loop(0, nvf, sc_body, 0)
        prevcur[pl.ds(0, 16)] = cursors
        pltpu.make_async_copy(outbuf, out_hbm.at[rj], sem_o).start()

    pltpu.make_async_copy(outbuf, out_hbm.at[r0 + _RPW - 1], sem_o).wait()


_sc_topk = pl.kernel(_sc_topk_body, **_KERNEL_KWARGS)


def kernel(x):
    return _sc_topk(x)
